# Initial kernel scaffold; baseline (speedup 1.0000x reference)
import jax
import jax.numpy as jnp
from jax.experimental import pallas as pl


def kernel(query_points, support_points, neighbors, x, K_points, K_values):
    raise NotImplementedError("write your pallas kernel here")


# speedup vs baseline: 1.3983x; 1.3983x over previous
import functools

import jax
import jax.numpy as jnp
from jax import lax
from jax.experimental import pallas as pl
from jax.experimental.pallas import tpu as pltpu
from jax.experimental.pallas import tpu_sc as plsc

N = 10000
H = 32
E = N * H
IN_F = 128
OUT_F = 128
N_KP = 15
KP_EXTENT = 1.0

NC, NS = 2, 16
NW = NC * NS
CH = 128
R_TOT = E // CH
R_BASE = R_TOT // NW
R_EXTRA = R_TOT % NW


def _sc_gather_body(feats_hbm, coords_hbm, idx_hbm, g_out, pt_out,
                    coords_v, idxbuf, gbuf, ptbuf,
                    tab_sem, idx_sem, gg_sem, wg_sem, wp_sem):
    wid = lax.axis_index("s") * NC + lax.axis_index("c")
    nch = jnp.where(wid < R_EXTRA, R_BASE + 1, R_BASE)
    row0 = R_BASE * wid + jnp.minimum(wid, R_EXTRA)

    pltpu.async_copy(coords_hbm, coords_v, tab_sem)

    def fire_idx(c, b):
        pltpu.async_copy(idx_hbm.at[row0 + c], idxbuf.at[b], idx_sem.at[b])

    def fire_gather(b):
        pltpu.async_copy(feats_hbm.at[idxbuf.at[b]], gbuf.at[b], gg_sem.at[b])

    def wait_gather(b):
        pltpu.make_async_copy(feats_hbm.at[idxbuf.at[b]], gbuf.at[b], gg_sem.at[b]).wait()

    def pack_coords(b):
        for j in range(CH // 16):
            idxv = idxbuf[b, pl.ds(j * 16, 16)] * 8
            for c in range(8):
                vals = plsc.load_gather(coords_v, [idxv + c])
                ptbuf[b, c, pl.ds(j * 16, 16)] = vals

    def fire_writebacks(c, b):
        pltpu.async_copy(gbuf.at[b], g_out.at[pl.ds((row0 + c) * CH, CH)], wg_sem.at[b])
        pltpu.async_copy(ptbuf.at[b], pt_out.at[pl.ds((row0 + c) * 8, 8)], wp_sem.at[b])

    def wait_writebacks(c, b):
        pltpu.make_async_copy(gbuf.at[b], g_out.at[pl.ds((row0 + c) * CH, CH)], wg_sem.at[b]).wait()
        pltpu.make_async_copy(ptbuf.at[b], pt_out.at[pl.ds((row0 + c) * 8, 8)], wp_sem.at[b]).wait()

    fire_idx(0, 0)
    fire_idx(1, 1)
    pltpu.make_async_copy(coords_hbm, coords_v, tab_sem).wait()

    @pl.loop(0, (nch + 1) // 2)
    def _outer(g):
        for b in range(2):
            c = g * 2 + b

            @pl.when(c < nch)
            def _chunk():
                pltpu.make_async_copy(idx_hbm.at[row0 + c], idxbuf.at[b], idx_sem.at[b]).wait()

                @pl.when(c >= 2)
                def _slot_free():
                    wait_writebacks(c, b)

                fire_gather(b)
                pack_coords(b)
                wait_gather(b)

                @pl.when(c + 2 < nch)
                def _prefetch():
                    fire_idx(c + 2, b)

                fire_writebacks(c, b)

    wait_writebacks(0, 0)
    wait_writebacks(0, 1)


def _make_sc_gather():
    mesh = plsc.VectorSubcoreMesh(core_axis_name="c", subcore_axis_name="s",
                                  num_cores=NC, num_subcores=NS)
    return pl.kernel(
        _sc_gather_body,
        out_type=[
            jax.ShapeDtypeStruct((E, IN_F), jnp.float32),
            jax.ShapeDtypeStruct((R_TOT * 8, 128), jnp.float32),
        ],
        mesh=mesh,
        compiler_params=pltpu.CompilerParams(needs_layout_passes=False),
        scratch_types=[
            pltpu.VMEM(((N + 1) * 8,), jnp.float32),
            pltpu.VMEM((2, CH), jnp.int32),
            pltpu.VMEM((2, CH, IN_F), jnp.float32),
            pltpu.VMEM((2, 8, 128), jnp.float32),
            pltpu.SemaphoreType.DMA,
            pltpu.SemaphoreType.DMA((2,)),
            pltpu.SemaphoreType.DMA((2,)),
            pltpu.SemaphoreType.DMA((2,)),
            pltpu.SemaphoreType.DMA((2,)),
        ],
    )


B = 200
EB = B * H


def _tc_body(g_ref, pt_ref, q_ref, c_ref, v_ref, o_ref):
    NCK = EB // CH
    Pt = pt_ref[...]
    P = jnp.swapaxes(Pt.reshape(NCK, 8, CH), 1, 2).reshape(EB, 8)
    Q = q_ref[...]
    Ev = (P.reshape(B, H, 8) - Q[:, None, :]).reshape(EB, 8)
    Ct = c_ref[...]
    EC = jnp.zeros((EB, 16), jnp.float32)
    C2 = jnp.zeros((1, 16), jnp.float32)
    for c in range(3):
        crow = Ct[c:c + 1, :]
        EC = EC + Ev[:, c:c + 1] * crow
        C2 = C2 + crow * crow
    E2 = jnp.sum(Ev * Ev, axis=1, keepdims=True)
    d2 = jnp.maximum(E2 - 2.0 * EC + C2, 0.0)
    W = jnp.maximum(1.0 - jnp.sqrt(d2) * (1.0 / KP_EXTENT), 0.0)

    G3 = g_ref[...].reshape(B, H, IN_F)
    W3 = W.reshape(B, H, 16)
    acc = jnp.zeros((B, OUT_F), jnp.float32)
    for k in range(N_KP):
        Fk = jnp.sum(W3[:, :, k:k + 1] * G3, axis=1)
        acc = acc + jnp.dot(Fk, v_ref[k], preferred_element_type=jnp.float32)
    o_ref[...] = acc


def _make_tc_compute():
    return pl.pallas_call(
        _tc_body,
        grid=(N // B,),
        in_specs=[
            pl.BlockSpec((EB, IN_F), lambda i: (i, 0)),
            pl.BlockSpec((EB // CH * 8, 128), lambda i: (i, 0)),
            pl.BlockSpec((B, 8), lambda i: (i, 0)),
            pl.BlockSpec((8, 16), lambda i: (0, 0)),
            pl.BlockSpec((N_KP, IN_F, OUT_F), lambda i: (0, 0, 0)),
        ],
        out_specs=pl.BlockSpec((B, OUT_F), lambda i: (i, 0)),
        out_shape=jax.ShapeDtypeStruct((N, OUT_F), jnp.float32),
    )


def kernel(query_points, support_points, neighbors, x, K_points, K_values):
    idx = jnp.where(neighbors < 0, N, neighbors).astype(jnp.int32).reshape(R_TOT, CH)
    feats_tab = jnp.concatenate([x, jnp.zeros((1, IN_F), x.dtype)], axis=0)
    coords_tab = jnp.concatenate(
        [support_points, jnp.full((1, 3), 1e6, support_points.dtype)], axis=0)
    coords_flat = jnp.pad(coords_tab, ((0, 0), (0, 5))).reshape(-1)
    q_pad = jnp.pad(query_points, ((0, 0), (0, 5)))
    c_pad = jnp.pad(K_points, ((0, 1), (0, 5)),
                    constant_values=0.0).at[N_KP, :3].set(1e6).T

    g, pt = _make_sc_gather()(feats_tab, coords_flat, idx)
    return _make_tc_compute()(g, pt, q_pad, c_pad, K_values)


# speedup vs baseline: 2.2541x; 1.6120x over previous
import functools

import jax
import jax.numpy as jnp
from jax import lax
from jax.experimental import pallas as pl
from jax.experimental.pallas import tpu as pltpu
from jax.experimental.pallas import tpu_sc as plsc

N = 10000
H = 32
E = N * H
IN_F = 128
OUT_F = 128
N_KP = 15
KP_EXTENT = 1.0

NC, NS = 2, 16
NW = NC * NS
CH = 128
R_TOT = E // CH
R_BASE = R_TOT // NW
R_EXTRA = R_TOT % NW


def _sc_gather_body(feats_hbm, coords_hbm, idx_hbm, g_out, pt_out,
                    coords_v, idxbuf, gbuf, ptbuf,
                    tab_sem, idx_sem, gg_sem, wg_sem, wp_sem):
    wid = lax.axis_index("s") * NC + lax.axis_index("c")
    nch = jnp.where(wid < R_EXTRA, R_BASE + 1, R_BASE)
    row0 = R_BASE * wid + jnp.minimum(wid, R_EXTRA)

    pltpu.async_copy(coords_hbm, coords_v, tab_sem)

    def fire_idx(c, b):
        pltpu.async_copy(idx_hbm.at[row0 + c], idxbuf.at[b], idx_sem.at[b])

    def fire_gather(b):
        pltpu.async_copy(feats_hbm.at[idxbuf.at[b]], gbuf.at[b], gg_sem.at[b])

    def wait_gather(b):
        pltpu.make_async_copy(feats_hbm.at[idxbuf.at[b]], gbuf.at[b], gg_sem.at[b]).wait()

    def pack_coords(b):
        for j in range(CH // 16):
            idxv = idxbuf[b, pl.ds(j * 16, 16)] * 8
            for c in range(8):
                vals = plsc.load_gather(coords_v, [idxv + c])
                ptbuf[b, c, pl.ds(j * 16, 16)] = vals

    def fire_writebacks(c, b):
        pltpu.async_copy(gbuf.at[b], g_out.at[pl.ds((row0 + c) * CH, CH)], wg_sem.at[b])
        pltpu.async_copy(ptbuf.at[b], pt_out.at[pl.ds((row0 + c) * 8, 8)], wp_sem.at[b])

    def wait_writebacks(c, b):
        pltpu.make_async_copy(gbuf.at[b], g_out.at[pl.ds((row0 + c) * CH, CH)], wg_sem.at[b]).wait()
        pltpu.make_async_copy(ptbuf.at[b], pt_out.at[pl.ds((row0 + c) * 8, 8)], wp_sem.at[b]).wait()

    fire_idx(0, 0)
    fire_idx(1, 1)
    pltpu.make_async_copy(coords_hbm, coords_v, tab_sem).wait()

    @pl.loop(0, (nch + 1) // 2)
    def _outer(g):
        for b in range(2):
            c = g * 2 + b

            @pl.when(c < nch)
            def _chunk():
                pltpu.make_async_copy(idx_hbm.at[row0 + c], idxbuf.at[b], idx_sem.at[b]).wait()

                @pl.when(c >= 2)
                def _slot_free():
                    wait_writebacks(c, b)

                fire_gather(b)
                pack_coords(b)
                wait_gather(b)

                @pl.when(c + 2 < nch)
                def _prefetch():
                    fire_idx(c + 2, b)

                fire_writebacks(c, b)

    wait_writebacks(0, 0)
    wait_writebacks(0, 1)


def _make_sc_gather():
    mesh = plsc.VectorSubcoreMesh(core_axis_name="c", subcore_axis_name="s",
                                  num_cores=NC, num_subcores=NS)
    return pl.kernel(
        _sc_gather_body,
        out_type=[
            jax.ShapeDtypeStruct((E, IN_F), jnp.float32),
            jax.ShapeDtypeStruct((R_TOT * 8, 128), jnp.float32),
        ],
        mesh=mesh,
        compiler_params=pltpu.CompilerParams(needs_layout_passes=False),
        scratch_types=[
            pltpu.VMEM(((N + 1) * 8,), jnp.float32),
            pltpu.VMEM((2, CH), jnp.int32),
            pltpu.VMEM((2, CH, IN_F), jnp.float32),
            pltpu.VMEM((2, 8, 128), jnp.float32),
            pltpu.SemaphoreType.DMA,
            pltpu.SemaphoreType.DMA((2,)),
            pltpu.SemaphoreType.DMA((2,)),
            pltpu.SemaphoreType.DMA((2,)),
            pltpu.SemaphoreType.DMA((2,)),
        ],
    )


B = 200
EB = B * H


def _tc_body(g_ref, pt_ref, q_ref, c_ref, v_ref, o_ref):
    NCK = EB // CH
    QPC = CH // H
    S3 = q_ref[...][:, None, :] + c_ref[...][None, :, :]
    SF = S3.reshape(B * 16, 8)
    S2 = jnp.sum(SF * SF, axis=1, keepdims=True)
    row_q = lax.broadcasted_iota(jnp.int32, (QPC * 16, CH), 0) // 16
    col_q = lax.broadcasted_iota(jnp.int32, (QPC * 16, CH), 1) // H
    bigmask = jnp.where(row_q == col_q, 0.0, 1e9).astype(jnp.float32)

    Pt3 = pt_ref[...].reshape(NCK, 8, CH)
    wf = []
    for m in range(NCK):
        Ptm = Pt3[m]
        P2 = jnp.sum(Ptm * Ptm, axis=0, keepdims=True)
        Sm = SF[m * QPC * 16:(m + 1) * QPC * 16, :]
        D = lax.dot_general(Sm, Ptm, (((1,), (0,)), ((), ())),
                            precision=lax.Precision.HIGHEST,
                            preferred_element_type=jnp.float32)
        d2 = P2 - 2.0 * D + S2[m * QPC * 16:(m + 1) * QPC * 16, :] + bigmask
        Wm = jnp.maximum(1.0 - jnp.sqrt(jnp.maximum(d2, 0.0)) * (1.0 / KP_EXTENT),
                         0.0)
        Gm = g_ref[pl.ds(m * CH, CH), :]
        wf.append(jnp.dot(Wm, Gm, preferred_element_type=jnp.float32))
    WF = jnp.stack(wf).reshape(NCK, QPC, 16, OUT_F)
    acc = jnp.zeros((B, OUT_F), jnp.float32)
    for k in range(N_KP):
        Fk = WF[:, :, k, :].reshape(B, IN_F)
        acc = acc + jnp.dot(Fk, v_ref[k], preferred_element_type=jnp.float32)
    o_ref[...] = acc


def _make_tc_compute():
    return pl.pallas_call(
        _tc_body,
        grid=(N // B,),
        in_specs=[
            pl.BlockSpec((EB, IN_F), lambda i: (i, 0)),
            pl.BlockSpec((EB // CH * 8, 128), lambda i: (i, 0)),
            pl.BlockSpec((B, 8), lambda i: (i, 0)),
            pl.BlockSpec((16, 8), lambda i: (0, 0)),
            pl.BlockSpec((N_KP, IN_F, OUT_F), lambda i: (0, 0, 0)),
        ],
        out_specs=pl.BlockSpec((B, OUT_F), lambda i: (i, 0)),
        out_shape=jax.ShapeDtypeStruct((N, OUT_F), jnp.float32),
    )


def kernel(query_points, support_points, neighbors, x, K_points, K_values):
    idx = jnp.where(neighbors < 0, N, neighbors).astype(jnp.int32).reshape(R_TOT, CH)
    feats_tab = jnp.concatenate([x, jnp.zeros((1, IN_F), x.dtype)], axis=0)
    coords_tab = jnp.concatenate(
        [support_points, jnp.full((1, 3), 1e6, support_points.dtype)], axis=0)
    coords_flat = jnp.pad(coords_tab, ((0, 0), (0, 5))).reshape(-1)
    q_pad = jnp.pad(query_points, ((0, 0), (0, 5)))
    c_pad = jnp.pad(K_points, ((0, 1), (0, 5)),
                    constant_values=0.0).at[N_KP, :3].set(1e6)

    g, pt = _make_sc_gather()(feats_tab, coords_flat, idx)
    return _make_tc_compute()(g, pt, q_pad, c_pad, K_values)


# speedup vs baseline: 4.3345x; 1.9229x over previous
import functools

import jax
import jax.numpy as jnp
from jax import lax
from jax.experimental import pallas as pl
from jax.experimental.pallas import tpu as pltpu
from jax.experimental.pallas import tpu_sc as plsc

N = 10000
H = 32
E = N * H
IN_F = 128
OUT_F = 128
N_KP = 15
KP_EXTENT = 1.0

NC, NS = 2, 16
NW = NC * NS
CH = 128
R_TOT = E // CH
R_BASE = R_TOT // NW
R_EXTRA = R_TOT % NW


def _sc_gather_body(feats_hbm, coords_hbm, idx_hbm, g_out, pt_out,
                    coords_v, idxbuf, gbuf, ptbuf,
                    tab_sem, idx_sem, gg_sem, wg_sem, wp_sem):
    wid = lax.axis_index("s") * NC + lax.axis_index("c")
    nch = jnp.where(wid < R_EXTRA, R_BASE + 1, R_BASE)
    row0 = R_BASE * wid + jnp.minimum(wid, R_EXTRA)

    pltpu.async_copy(coords_hbm, coords_v, tab_sem)

    def fire_idx(c, b):
        pltpu.async_copy(idx_hbm.at[row0 + c], idxbuf.at[b], idx_sem.at[b])

    def fire_gather(b):
        pltpu.async_copy(feats_hbm.at[idxbuf.at[b]], gbuf.at[b], gg_sem.at[b])

    def wait_gather(b):
        pltpu.make_async_copy(feats_hbm.at[idxbuf.at[b]], gbuf.at[b], gg_sem.at[b]).wait()

    def pack_coords(b):
        for j in range(CH // 16):
            idxv = idxbuf[b, pl.ds(j * 16, 16)] * 8
            for c in range(8):
                vals = plsc.load_gather(coords_v, [idxv + c])
                ptbuf[b, c, pl.ds(j * 16, 16)] = vals

    def fire_writebacks(c, b):
        pltpu.async_copy(gbuf.at[b], g_out.at[pl.ds((row0 + c) * CH, CH)], wg_sem.at[b])
        pltpu.async_copy(ptbuf.at[b], pt_out.at[pl.ds((row0 + c) * 8, 8)], wp_sem.at[b])

    def wait_writebacks(c, b):
        pltpu.make_async_copy(gbuf.at[b], g_out.at[pl.ds((row0 + c) * CH, CH)], wg_sem.at[b]).wait()
        pltpu.make_async_copy(ptbuf.at[b], pt_out.at[pl.ds((row0 + c) * 8, 8)], wp_sem.at[b]).wait()

    fire_idx(0, 0)
    fire_idx(1, 1)
    pltpu.make_async_copy(coords_hbm, coords_v, tab_sem).wait()

    @pl.loop(0, (nch + 1) // 2)
    def _outer(g):
        for b in range(2):
            c = g * 2 + b

            @pl.when(c < nch)
            def _chunk():
                pltpu.make_async_copy(idx_hbm.at[row0 + c], idxbuf.at[b], idx_sem.at[b]).wait()

                @pl.when(c >= 2)
                def _slot_free():
                    wait_writebacks(c, b)

                fire_gather(b)
                pack_coords(b)
                wait_gather(b)

                @pl.when(c + 2 < nch)
                def _prefetch():
                    fire_idx(c + 2, b)

                fire_writebacks(c, b)

    wait_writebacks(0, 0)
    wait_writebacks(0, 1)


def _make_sc_gather():
    mesh = plsc.VectorSubcoreMesh(core_axis_name="c", subcore_axis_name="s",
                                  num_cores=NC, num_subcores=NS)
    return pl.kernel(
        _sc_gather_body,
        out_type=[
            jax.ShapeDtypeStruct((E, IN_F), jnp.float32),
            jax.ShapeDtypeStruct((R_TOT * 8, 128), jnp.float32),
        ],
        mesh=mesh,
        compiler_params=pltpu.CompilerParams(needs_layout_passes=False),
        scratch_types=[
            pltpu.VMEM(((N + 1) * 8,), jnp.float32),
            pltpu.VMEM((2, CH), jnp.int32),
            pltpu.VMEM((2, CH, IN_F), jnp.float32),
            pltpu.VMEM((2, 8, 128), jnp.float32),
            pltpu.SemaphoreType.DMA,
            pltpu.SemaphoreType.DMA((2,)),
            pltpu.SemaphoreType.DMA((2,)),
            pltpu.SemaphoreType.DMA((2,)),
            pltpu.SemaphoreType.DMA((2,)),
        ],
    )


B = 200
EB = B * H


def _tc_body(g_ref, pt_ref, q_ref, c_ref, v_ref, o_ref):
    NCK = EB // CH
    QPC = CH // H
    S3 = q_ref[...][:, None, :] + c_ref[...][None, :, :]
    SF = S3.reshape(B * 16, 8)
    S2 = jnp.sum(SF * SF, axis=1, keepdims=True)
    row_q = lax.broadcasted_iota(jnp.int32, (QPC * 16, CH), 0) // 16
    col_q = lax.broadcasted_iota(jnp.int32, (QPC * 16, CH), 1) // H
    bigmask = jnp.where(row_q == col_q, 0.0, 1e9).astype(jnp.float32)

    Pt3 = pt_ref[...].reshape(NCK, 8, CH)
    wf = []
    for m in range(NCK):
        Ptm = Pt3[m]
        Sm = SF[m * QPC * 16:(m + 1) * QPC * 16, :]
        d2 = S2[m * QPC * 16:(m + 1) * QPC * 16, :] + bigmask
        for c in range(3):
            prow = Ptm[c:c + 1, :]
            d2 = d2 + prow * (prow - 2.0 * Sm[:, c:c + 1])
        Wm = jnp.maximum(1.0 - jnp.sqrt(jnp.maximum(d2, 0.0)) * (1.0 / KP_EXTENT),
                         0.0)
        Gm = g_ref[m * CH:(m + 1) * CH, :]
        wf.append(jnp.dot(Wm, Gm, preferred_element_type=jnp.float32))
    WF = jnp.stack(wf).reshape(NCK, QPC, 16, OUT_F)
    acc = jnp.zeros((B, OUT_F), jnp.float32)
    for k in range(N_KP):
        Fk = WF[:, :, k, :].reshape(B, IN_F)
        acc = acc + jnp.dot(Fk, v_ref[k], preferred_element_type=jnp.float32)
    o_ref[...] = acc


def _make_tc_compute():
    return pl.pallas_call(
        _tc_body,
        grid=(N // B,),
        in_specs=[
            pl.BlockSpec((EB, IN_F), lambda i: (i, 0)),
            pl.BlockSpec((EB // CH * 8, 128), lambda i: (i, 0)),
            pl.BlockSpec((B, 8), lambda i: (i, 0)),
            pl.BlockSpec((16, 8), lambda i: (0, 0)),
            pl.BlockSpec((N_KP, IN_F, OUT_F), lambda i: (0, 0, 0)),
        ],
        out_specs=pl.BlockSpec((B, OUT_F), lambda i: (i, 0)),
        out_shape=jax.ShapeDtypeStruct((N, OUT_F), jnp.float32),
    )


def kernel(query_points, support_points, neighbors, x, K_points, K_values):
    idx = jnp.where(neighbors < 0, N, neighbors).astype(jnp.int32).reshape(R_TOT, CH)
    feats_tab = jnp.concatenate([x, jnp.zeros((1, IN_F), x.dtype)], axis=0)
    coords_tab = jnp.concatenate(
        [support_points, jnp.full((1, 3), 1e6, support_points.dtype)], axis=0)
    coords_flat = jnp.pad(coords_tab, ((0, 0), (0, 5))).reshape(-1)
    q_pad = jnp.pad(query_points, ((0, 0), (0, 5)))
    c_pad = jnp.pad(K_points, ((0, 1), (0, 5)),
                    constant_values=0.0).at[N_KP, :3].set(1e6)

    g, pt = _make_sc_gather()(feats_tab, coords_flat, idx)
    return _make_tc_compute()(g, pt, q_pad, c_pad, K_values)


# speedup vs baseline: 4.5249x; 1.0439x over previous
import functools

import jax
import jax.numpy as jnp
from jax import lax
from jax.experimental import pallas as pl
from jax.experimental.pallas import tpu as pltpu
from jax.experimental.pallas import tpu_sc as plsc

N = 10000
H = 32
E = N * H
IN_F = 128
OUT_F = 128
N_KP = 15
KP_EXTENT = 1.0

NC, NS = 2, 16
NW = NC * NS
CH = 128
R_TOT = E // CH
R_BASE = R_TOT // NW
R_EXTRA = R_TOT % NW


def _sc_gather_body(r_slice, feats_hbm, coords_hbm, idx_hbm, g_out, pt_out,
                    coords_v, idxbuf, gbuf, ptbuf,
                    tab_sem, idx_sem, gg_sem, wg_sem, wp_sem):
    r_base, r_extra = r_slice // NW, r_slice % NW
    wid = lax.axis_index("s") * NC + lax.axis_index("c")
    nch = jnp.where(wid < r_extra, r_base + 1, r_base)
    row0 = r_base * wid + jnp.minimum(wid, r_extra)

    pltpu.async_copy(coords_hbm, coords_v, tab_sem)

    def fire_idx(c, b):
        pltpu.async_copy(idx_hbm.at[row0 + c], idxbuf.at[b], idx_sem.at[b])

    def fire_gather(b):
        pltpu.async_copy(feats_hbm.at[idxbuf.at[b]], gbuf.at[b], gg_sem.at[b])

    def wait_gather(b):
        pltpu.make_async_copy(feats_hbm.at[idxbuf.at[b]], gbuf.at[b], gg_sem.at[b]).wait()

    def pack_coords(b):
        for j in range(CH // 16):
            idxv = idxbuf[b, pl.ds(j * 16, 16)] * 8
            for c in range(8):
                vals = plsc.load_gather(coords_v, [idxv + c])
                ptbuf[b, c, pl.ds(j * 16, 16)] = vals

    def fire_writebacks(c, b):
        pltpu.async_copy(gbuf.at[b], g_out.at[pl.ds((row0 + c) * CH, CH)], wg_sem.at[b])
        pltpu.async_copy(ptbuf.at[b], pt_out.at[pl.ds((row0 + c) * 8, 8)], wp_sem.at[b])

    def wait_writebacks(c, b):
        pltpu.make_async_copy(gbuf.at[b], g_out.at[pl.ds((row0 + c) * CH, CH)], wg_sem.at[b]).wait()
        pltpu.make_async_copy(ptbuf.at[b], pt_out.at[pl.ds((row0 + c) * 8, 8)], wp_sem.at[b]).wait()

    fire_idx(0, 0)
    fire_idx(1, 1)
    pltpu.make_async_copy(coords_hbm, coords_v, tab_sem).wait()

    @pl.loop(0, (nch + 1) // 2)
    def _outer(g):
        for b in range(2):
            c = g * 2 + b

            @pl.when(c < nch)
            def _chunk():
                pltpu.make_async_copy(idx_hbm.at[row0 + c], idxbuf.at[b], idx_sem.at[b]).wait()

                @pl.when(c >= 2)
                def _slot_free():
                    wait_writebacks(c, b)

                fire_gather(b)
                pack_coords(b)
                wait_gather(b)

                @pl.when(c + 2 < nch)
                def _prefetch():
                    fire_idx(c + 2, b)

                fire_writebacks(c, b)

    wait_writebacks(0, 0)
    wait_writebacks(0, 1)


def _make_sc_gather(r_slice):
    mesh = plsc.VectorSubcoreMesh(core_axis_name="c", subcore_axis_name="s",
                                  num_cores=NC, num_subcores=NS)
    return pl.kernel(
        functools.partial(_sc_gather_body, r_slice),
        out_type=[
            jax.ShapeDtypeStruct((r_slice * CH, IN_F), jnp.float32),
            jax.ShapeDtypeStruct((r_slice * 8, 128), jnp.float32),
        ],
        mesh=mesh,
        compiler_params=pltpu.CompilerParams(needs_layout_passes=False),
        scratch_types=[
            pltpu.VMEM(((N + 1) * 8,), jnp.float32),
            pltpu.VMEM((2, CH), jnp.int32),
            pltpu.VMEM((2, CH, IN_F), jnp.float32),
            pltpu.VMEM((2, 8, 128), jnp.float32),
            pltpu.SemaphoreType.DMA,
            pltpu.SemaphoreType.DMA((2,)),
            pltpu.SemaphoreType.DMA((2,)),
            pltpu.SemaphoreType.DMA((2,)),
            pltpu.SemaphoreType.DMA((2,)),
        ],
    )


B = 200
EB = B * H


def _tc_body(g_ref, pt_ref, q_ref, c_ref, v_ref, o_ref):
    NCK = EB // CH
    QPC = CH // H
    S3 = q_ref[...][:, None, :] + c_ref[...][None, :, :]
    SF = S3.reshape(B * 16, 8)
    S2 = jnp.sum(SF * SF, axis=1, keepdims=True)
    row_q = lax.broadcasted_iota(jnp.int32, (QPC * 16, CH), 0) // 16
    col_q = lax.broadcasted_iota(jnp.int32, (QPC * 16, CH), 1) // H
    bigmask = jnp.where(row_q == col_q, 0.0, 1e9).astype(jnp.float32)

    Pt3 = pt_ref[...].reshape(NCK, 8, CH)
    wf = []
    for m in range(NCK):
        Ptm = Pt3[m]
        Sm = SF[m * QPC * 16:(m + 1) * QPC * 16, :]
        d2 = S2[m * QPC * 16:(m + 1) * QPC * 16, :] + bigmask
        for c in range(3):
            prow = Ptm[c:c + 1, :]
            d2 = d2 + prow * (prow - 2.0 * Sm[:, c:c + 1])
        Wm = jnp.maximum(1.0 - jnp.sqrt(jnp.maximum(d2, 0.0)) * (1.0 / KP_EXTENT),
                         0.0)
        Gm = g_ref[m * CH:(m + 1) * CH, :]
        wf.append(jnp.dot(Wm, Gm, preferred_element_type=jnp.float32))
    WF = jnp.stack(wf).reshape(NCK, QPC, 16, OUT_F)
    acc = jnp.zeros((B, OUT_F), jnp.float32)
    for k in range(N_KP):
        Fk = WF[:, :, k, :].reshape(B, IN_F)
        acc = acc + jnp.dot(Fk, v_ref[k], preferred_element_type=jnp.float32)
    o_ref[...] = acc


def _make_tc_compute(nq):
    return pl.pallas_call(
        _tc_body,
        grid=(nq // B,),
        in_specs=[
            pl.BlockSpec((EB, IN_F), lambda i: (i, 0)),
            pl.BlockSpec((EB // CH * 8, 128), lambda i: (i, 0)),
            pl.BlockSpec((B, 8), lambda i: (i, 0)),
            pl.BlockSpec((16, 8), lambda i: (0, 0)),
            pl.BlockSpec((N_KP, IN_F, OUT_F), lambda i: (0, 0, 0)),
        ],
        out_specs=pl.BlockSpec((B, OUT_F), lambda i: (i, 0)),
        out_shape=jax.ShapeDtypeStruct((nq, OUT_F), jnp.float32),
    )


def kernel(query_points, support_points, neighbors, x, K_points, K_values):
    idx = jnp.where(neighbors < 0, N, neighbors).astype(jnp.int32).reshape(R_TOT, CH)
    feats_tab = jnp.concatenate([x, jnp.zeros((1, IN_F), x.dtype)], axis=0)
    coords_tab = jnp.concatenate(
        [support_points, jnp.full((1, 3), 1e6, support_points.dtype)], axis=0)
    coords_flat = jnp.pad(coords_tab, ((0, 0), (0, 5))).reshape(-1)
    q_pad = jnp.pad(query_points, ((0, 0), (0, 5)))
    c_pad = jnp.pad(K_points, ((0, 1), (0, 5)),
                    constant_values=0.0).at[N_KP, :3].set(1e6)

    NSLC = 5
    r_slice = R_TOT // NSLC
    nq = N // NSLC
    sc_call = _make_sc_gather(r_slice)
    tc_call = _make_tc_compute(nq)
    outs = []
    for sl in range(NSLC):
        g, pt = sc_call(feats_tab, coords_flat,
                        lax.slice_in_dim(idx, sl * r_slice, (sl + 1) * r_slice))
        outs.append(tc_call(g, pt,
                            lax.slice_in_dim(q_pad, sl * nq, (sl + 1) * nq),
                            c_pad, K_values))
    return jnp.concatenate(outs, axis=0)


# speedup vs baseline: 4.6323x; 1.0237x over previous
import functools

import jax
import jax.numpy as jnp
from jax import lax
from jax.experimental import pallas as pl
from jax.experimental.pallas import tpu as pltpu
from jax.experimental.pallas import tpu_sc as plsc

N = 10000
H = 32
E = N * H
IN_F = 128
OUT_F = 128
N_KP = 15
KP_EXTENT = 1.0

NC, NS = 2, 16
NW = NC * NS
CH = 128
R_TOT = E // CH
R_BASE = R_TOT // NW
R_EXTRA = R_TOT % NW


def _sc_gather_body(r_slice, feats_hbm, coords_hbm, idx_hbm, g_out, pt_out,
                    coords_v, idxbuf, gbuf, ptbuf,
                    tab_sem, idx_sem, gg_sem, wg_sem, wp_sem):
    r_base, r_extra = r_slice // NW, r_slice % NW
    wid = lax.axis_index("s") * NC + lax.axis_index("c")
    nch = jnp.where(wid < r_extra, r_base + 1, r_base)
    row0 = r_base * wid + jnp.minimum(wid, r_extra)

    pltpu.async_copy(coords_hbm, coords_v, tab_sem)

    def fire_idx(c, b):
        pltpu.async_copy(idx_hbm.at[row0 + c], idxbuf.at[b], idx_sem.at[b])

    def fire_gather(b):
        pltpu.async_copy(feats_hbm.at[idxbuf.at[b]], gbuf.at[b], gg_sem.at[b])

    def wait_gather(b):
        pltpu.make_async_copy(feats_hbm.at[idxbuf.at[b]], gbuf.at[b], gg_sem.at[b]).wait()

    def pack_coords(b):
        for j in range(CH // 16):
            idxv = idxbuf[b, pl.ds(j * 16, 16)] * 8
            for c in range(8):
                vals = plsc.load_gather(coords_v, [idxv + c])
                ptbuf[b, c, pl.ds(j * 16, 16)] = vals

    def fire_writebacks(c, b):
        pltpu.async_copy(gbuf.at[b], g_out.at[pl.ds((row0 + c) * CH, CH)], wg_sem.at[b])
        pltpu.async_copy(ptbuf.at[b], pt_out.at[pl.ds((row0 + c) * 8, 8)], wp_sem.at[b])

    def wait_writebacks(c, b):
        pltpu.make_async_copy(gbuf.at[b], g_out.at[pl.ds((row0 + c) * CH, CH)], wg_sem.at[b]).wait()
        pltpu.make_async_copy(ptbuf.at[b], pt_out.at[pl.ds((row0 + c) * 8, 8)], wp_sem.at[b]).wait()

    fire_idx(0, 0)
    fire_idx(1, 1)
    pltpu.make_async_copy(coords_hbm, coords_v, tab_sem).wait()

    @pl.loop(0, (nch + 1) // 2)
    def _outer(g):
        for b in range(2):
            c = g * 2 + b

            @pl.when(c < nch)
            def _chunk():
                pltpu.make_async_copy(idx_hbm.at[row0 + c], idxbuf.at[b], idx_sem.at[b]).wait()

                @pl.when(c >= 2)
                def _slot_free():
                    wait_writebacks(c, b)

                fire_gather(b)
                pack_coords(b)
                wait_gather(b)

                @pl.when(c + 2 < nch)
                def _prefetch():
                    fire_idx(c + 2, b)

                fire_writebacks(c, b)

    wait_writebacks(0, 0)
    wait_writebacks(0, 1)


def _make_sc_gather(r_slice):
    mesh = plsc.VectorSubcoreMesh(core_axis_name="c", subcore_axis_name="s",
                                  num_cores=NC, num_subcores=NS)
    return pl.kernel(
        functools.partial(_sc_gather_body, r_slice),
        out_type=[
            jax.ShapeDtypeStruct((r_slice * CH, IN_F), jnp.float32),
            jax.ShapeDtypeStruct((r_slice * 8, 128), jnp.float32),
        ],
        mesh=mesh,
        compiler_params=pltpu.CompilerParams(needs_layout_passes=False),
        scratch_types=[
            pltpu.VMEM(((N + 1) * 8,), jnp.float32),
            pltpu.VMEM((2, CH), jnp.int32),
            pltpu.VMEM((2, CH, IN_F), jnp.float32),
            pltpu.VMEM((2, 8, 128), jnp.float32),
            pltpu.SemaphoreType.DMA,
            pltpu.SemaphoreType.DMA((2,)),
            pltpu.SemaphoreType.DMA((2,)),
            pltpu.SemaphoreType.DMA((2,)),
            pltpu.SemaphoreType.DMA((2,)),
        ],
    )


B = 400
EB = B * H


def _tc_body(g_ref, pt_ref, q_ref, c_ref, v_ref, o_ref):
    NCK = EB // CH
    QPC = CH // H
    S3 = q_ref[...][:, None, :] + c_ref[...][None, :, :]
    SF = S3.reshape(B * 16, 8)
    S2 = jnp.sum(SF * SF, axis=1, keepdims=True)
    row_q = lax.broadcasted_iota(jnp.int32, (QPC * 16, CH), 0) // 16
    col_q = lax.broadcasted_iota(jnp.int32, (QPC * 16, CH), 1) // H
    bigmask = jnp.where(row_q == col_q, 0.0, 1e9).astype(jnp.float32)

    Pt3 = pt_ref[...].reshape(NCK, 8, CH)
    wf = []
    for m in range(NCK):
        Ptm = Pt3[m]
        Sm = SF[m * QPC * 16:(m + 1) * QPC * 16, :]
        d2 = S2[m * QPC * 16:(m + 1) * QPC * 16, :] + bigmask
        for c in range(3):
            prow = Ptm[c:c + 1, :]
            d2 = d2 + prow * (prow - 2.0 * Sm[:, c:c + 1])
        Wm = jnp.maximum(1.0 - jnp.sqrt(jnp.maximum(d2, 0.0)) * (1.0 / KP_EXTENT),
                         0.0)
        Gm = g_ref[m * CH:(m + 1) * CH, :]
        wf.append(jnp.dot(Wm, Gm, preferred_element_type=jnp.float32))
    WF = jnp.stack(wf).reshape(NCK, QPC, 16, OUT_F)
    acc = jnp.zeros((B, OUT_F), jnp.float32)
    for k in range(N_KP):
        Fk = WF[:, :, k, :].reshape(B, IN_F)
        acc = acc + jnp.dot(Fk, v_ref[k], preferred_element_type=jnp.float32)
    o_ref[...] = acc


def _make_tc_compute(nq):
    return pl.pallas_call(
        _tc_body,
        grid=(nq // B,),
        in_specs=[
            pl.BlockSpec((EB, IN_F), lambda i: (i, 0)),
            pl.BlockSpec((EB // CH * 8, 128), lambda i: (i, 0)),
            pl.BlockSpec((B, 8), lambda i: (i, 0)),
            pl.BlockSpec((16, 8), lambda i: (0, 0)),
            pl.BlockSpec((N_KP, IN_F, OUT_F), lambda i: (0, 0, 0)),
        ],
        out_specs=pl.BlockSpec((B, OUT_F), lambda i: (i, 0)),
        out_shape=jax.ShapeDtypeStruct((nq, OUT_F), jnp.float32),
    )


def kernel(query_points, support_points, neighbors, x, K_points, K_values):
    idx = jnp.where(neighbors < 0, N, neighbors).astype(jnp.int32).reshape(R_TOT, CH)
    feats_tab = jnp.concatenate([x, jnp.zeros((1, IN_F), x.dtype)], axis=0)
    coords_tab = jnp.concatenate(
        [support_points, jnp.full((1, 3), 1e6, support_points.dtype)], axis=0)
    coords_flat = jnp.pad(coords_tab, ((0, 0), (0, 5))).reshape(-1)
    q_pad = jnp.pad(query_points, ((0, 0), (0, 5)))
    c_pad = jnp.pad(K_points, ((0, 1), (0, 5)),
                    constant_values=0.0).at[N_KP, :3].set(1e6)

    NSLC = 5
    r_slice = R_TOT // NSLC
    nq = N // NSLC
    sc_call = _make_sc_gather(r_slice)
    tc_call = _make_tc_compute(nq)
    outs = []
    for sl in range(NSLC):
        g, pt = sc_call(feats_tab, coords_flat,
                        lax.slice_in_dim(idx, sl * r_slice, (sl + 1) * r_slice))
        outs.append(tc_call(g, pt,
                            lax.slice_in_dim(q_pad, sl * nq, (sl + 1) * nq),
                            c_pad, K_values))
    return jnp.concatenate(outs, axis=0)


# speedup vs baseline: 4.7788x; 1.0316x over previous
import functools

import jax
import jax.numpy as jnp
from jax import lax
from jax.experimental import pallas as pl
from jax.experimental.pallas import tpu as pltpu
from jax.experimental.pallas import tpu_sc as plsc

N = 10000
H = 32
E = N * H
IN_F = 128
OUT_F = 128
N_KP = 15
KP_EXTENT = 1.0

NC, NS = 2, 16
NW = NC * NS
CH = 128
R_TOT = E // CH
R_BASE = R_TOT // NW
R_EXTRA = R_TOT % NW


def _sc_pack_coords_body(coords_hbm, idx_hbm, pt_out,
                         coords_v, idxbuf, ptbuf, tab_sem, idx_sem, wp_sem):
    wid = lax.axis_index("s") * NC + lax.axis_index("c")
    r_base, r_extra = R_TOT // NW, R_TOT % NW
    nch = jnp.where(wid < r_extra, r_base + 1, r_base)
    row0 = r_base * wid + jnp.minimum(wid, r_extra)

    pltpu.async_copy(coords_hbm, coords_v, tab_sem)

    def fire_idx(c, b):
        pltpu.async_copy(idx_hbm.at[row0 + c], idxbuf.at[b], idx_sem.at[b])

    def pack(b):
        for j in range(CH // 16):
            idxv = idxbuf[b, pl.ds(j * 16, 16)] * 8
            for cc in range(8):
                ptbuf[b, cc, pl.ds(j * 16, 16)] = plsc.load_gather(
                    coords_v, [idxv + cc])

    def fire_wb(c, b):
        pltpu.async_copy(ptbuf.at[b], pt_out.at[pl.ds((row0 + c) * 8, 8)], wp_sem.at[b])

    def wait_wb(c, b):
        pltpu.make_async_copy(ptbuf.at[b], pt_out.at[pl.ds((row0 + c) * 8, 8)], wp_sem.at[b]).wait()

    fire_idx(0, 0)
    fire_idx(1, 1)
    pltpu.make_async_copy(coords_hbm, coords_v, tab_sem).wait()

    @pl.loop(0, (nch + 1) // 2)
    def _outer(g):
        for b in range(2):
            c = g * 2 + b

            @pl.when(c < nch)
            def _chunk():
                pltpu.make_async_copy(idx_hbm.at[row0 + c], idxbuf.at[b], idx_sem.at[b]).wait()

                @pl.when(c >= 2)
                def _slot_free():
                    wait_wb(c, b)

                pack(b)

                @pl.when(c + 2 < nch)
                def _prefetch():
                    fire_idx(c + 2, b)

                fire_wb(c, b)

    wait_wb(0, 0)
    wait_wb(0, 1)


def _sc_feat_body(r_slice, feats_hbm, idx_hbm, g_out,
                  idxbuf, gbuf, idx_sem, gg_sem, wg_sem):
    r_base, r_extra = r_slice // NW, r_slice % NW
    wid = lax.axis_index("s") * NC + lax.axis_index("c")
    nch = jnp.where(wid < r_extra, r_base + 1, r_base)
    row0 = r_base * wid + jnp.minimum(wid, r_extra)

    def fire_idx(c, b):
        pltpu.async_copy(idx_hbm.at[row0 + c], idxbuf.at[b], idx_sem.at[b])

    def fire_gather(b):
        pltpu.async_copy(feats_hbm.at[idxbuf.at[b]], gbuf.at[b], gg_sem.at[b])

    def wait_gather(b):
        pltpu.make_async_copy(feats_hbm.at[idxbuf.at[b]], gbuf.at[b], gg_sem.at[b]).wait()

    def fire_wb(c, b):
        pltpu.async_copy(gbuf.at[b], g_out.at[pl.ds((row0 + c) * CH, CH)], wg_sem.at[b])

    def wait_wb(c, b):
        pltpu.make_async_copy(gbuf.at[b], g_out.at[pl.ds((row0 + c) * CH, CH)], wg_sem.at[b]).wait()

    fire_idx(0, 0)
    fire_idx(1, 1)

    @pl.loop(0, (nch + 1) // 2)
    def _outer(g):
        for b in range(2):
            c = g * 2 + b

            @pl.when(c < nch)
            def _chunk():
                pltpu.make_async_copy(idx_hbm.at[row0 + c], idxbuf.at[b], idx_sem.at[b]).wait()

                @pl.when(c >= 2)
                def _slot_free():
                    wait_wb(c, b)

                fire_gather(b)
                wait_gather(b)

                @pl.when(c + 2 < nch)
                def _prefetch():
                    fire_idx(c + 2, b)

                fire_wb(c, b)

    wait_wb(0, 0)
    wait_wb(0, 1)


def _sc_mesh():
    return plsc.VectorSubcoreMesh(core_axis_name="c", subcore_axis_name="s",
                                  num_cores=NC, num_subcores=NS)


def _make_sc_pack_coords():
    return pl.kernel(
        _sc_pack_coords_body,
        out_type=jax.ShapeDtypeStruct((R_TOT * 8, 128), jnp.float32),
        mesh=_sc_mesh(),
        compiler_params=pltpu.CompilerParams(needs_layout_passes=False),
        scratch_types=[
            pltpu.VMEM(((N + 1) * 8,), jnp.float32),
            pltpu.VMEM((2, CH), jnp.int32),
            pltpu.VMEM((2, 8, 128), jnp.float32),
            pltpu.SemaphoreType.DMA,
            pltpu.SemaphoreType.DMA((2,)),
            pltpu.SemaphoreType.DMA((2,)),
        ],
    )


def _make_sc_feat(r_slice):
    return pl.kernel(
        functools.partial(_sc_feat_body, r_slice),
        out_type=jax.ShapeDtypeStruct((r_slice * CH, IN_F), jnp.float32),
        mesh=_sc_mesh(),
        compiler_params=pltpu.CompilerParams(needs_layout_passes=False),
        scratch_types=[
            pltpu.VMEM((2, CH), jnp.int32),
            pltpu.VMEM((2, CH, IN_F), jnp.float32),
            pltpu.SemaphoreType.DMA((2,)),
            pltpu.SemaphoreType.DMA((2,)),
            pltpu.SemaphoreType.DMA((2,)),
        ],
    )


B = 400
EB = B * H


def _tc_body(g_ref, pt_ref, q_ref, c_ref, v_ref, o_ref):
    NCK = EB // CH
    QPC = CH // H
    S3 = q_ref[...][:, None, :] + c_ref[...][None, :, :]
    SF = S3.reshape(B * 16, 8)
    S2 = jnp.sum(SF * SF, axis=1, keepdims=True)
    row_q = lax.broadcasted_iota(jnp.int32, (QPC * 16, CH), 0) // 16
    col_q = lax.broadcasted_iota(jnp.int32, (QPC * 16, CH), 1) // H
    bigmask = jnp.where(row_q == col_q, 0.0, 1e9).astype(jnp.float32)

    Pt3 = pt_ref[...].reshape(NCK, 8, CH)
    wf = []
    for m in range(NCK):
        Ptm = Pt3[m]
        Sm = SF[m * QPC * 16:(m + 1) * QPC * 16, :]
        d2 = S2[m * QPC * 16:(m + 1) * QPC * 16, :] + bigmask
        for c in range(3):
            prow = Ptm[c:c + 1, :]
            d2 = d2 + prow * (prow - 2.0 * Sm[:, c:c + 1])
        Wm = jnp.maximum(1.0 - jnp.sqrt(jnp.maximum(d2, 0.0)) * (1.0 / KP_EXTENT),
                         0.0)
        Gm = g_ref[m * CH:(m + 1) * CH, :]
        wf.append(jnp.dot(Wm, Gm, preferred_element_type=jnp.float32))
    WF = jnp.stack(wf).reshape(NCK, QPC, 16, OUT_F)
    acc = jnp.zeros((B, OUT_F), jnp.float32)
    for k in range(N_KP):
        Fk = WF[:, :, k, :].reshape(B, IN_F)
        acc = acc + jnp.dot(Fk, v_ref[k], preferred_element_type=jnp.float32)
    o_ref[...] = acc


def _make_tc_compute(nq, sl):
    nblk = nq // B
    off = sl * nblk
    return pl.pallas_call(
        _tc_body,
        grid=(nblk,),
        in_specs=[
            pl.BlockSpec((EB, IN_F), lambda i: (i, 0)),
            pl.BlockSpec((EB // CH * 8, 128), lambda i: (i + off, 0)),
            pl.BlockSpec((B, 8), lambda i: (i + off, 0)),
            pl.BlockSpec((16, 8), lambda i: (0, 0)),
            pl.BlockSpec((N_KP, IN_F, OUT_F), lambda i: (0, 0, 0)),
        ],
        out_specs=pl.BlockSpec((B, OUT_F), lambda i: (i, 0)),
        out_shape=jax.ShapeDtypeStruct((nq, OUT_F), jnp.float32),
    )


def kernel(query_points, support_points, neighbors, x, K_points, K_values):
    idx = jnp.where(neighbors < 0, N, neighbors).astype(jnp.int32).reshape(R_TOT, CH)
    feats_tab = jnp.concatenate([x, jnp.zeros((1, IN_F), x.dtype)], axis=0)
    coords_tab = jnp.concatenate(
        [support_points, jnp.full((1, 3), 1e6, support_points.dtype)], axis=0)
    coords_flat = jnp.pad(coords_tab, ((0, 0), (0, 5))).reshape(-1)
    q_pad = jnp.pad(query_points, ((0, 0), (0, 5)))
    c_pad = jnp.pad(K_points, ((0, 1), (0, 5)),
                    constant_values=0.0).at[N_KP, :3].set(1e6)

    NSLC = 5
    r_slice = R_TOT // NSLC
    nq = N // NSLC
    pt = _make_sc_pack_coords()(coords_flat, idx)
    sc_call = _make_sc_feat(r_slice)
    outs = []
    for sl in range(NSLC):
        g = sc_call(feats_tab,
                    lax.slice_in_dim(idx, sl * r_slice, (sl + 1) * r_slice))
        outs.append(_make_tc_compute(nq, sl)(g, pt, q_pad, c_pad, K_values))
    return jnp.concatenate(outs, axis=0)


# speedup vs baseline: 4.9158x; 1.0287x over previous
import functools

import jax
import jax.numpy as jnp
from jax import lax
from jax.experimental import pallas as pl
from jax.experimental.pallas import tpu as pltpu
from jax.experimental.pallas import tpu_sc as plsc

N = 10000
H = 32
E = N * H
IN_F = 128
OUT_F = 128
N_KP = 15
KP_EXTENT = 1.0

NC, NS = 2, 16
NW = NC * NS
CH = 128
R_TOT = E // CH
R_BASE = R_TOT // NW
R_EXTRA = R_TOT % NW


def _sc_pack_coords_body(coords_hbm, idx_hbm, pt_out,
                         coords_v, idxbuf, ptbuf, tab_sem, idx_sem, wp_sem):
    wid = lax.axis_index("s") * NC + lax.axis_index("c")
    r_base, r_extra = R_TOT // NW, R_TOT % NW
    nch = jnp.where(wid < r_extra, r_base + 1, r_base)
    row0 = r_base * wid + jnp.minimum(wid, r_extra)

    pltpu.async_copy(coords_hbm, coords_v, tab_sem)

    def fire_idx(c, b):
        pltpu.async_copy(idx_hbm.at[row0 + c], idxbuf.at[b], idx_sem.at[b])

    def pack(b):
        for j in range(CH // 16):
            idxv = idxbuf[b, pl.ds(j * 16, 16)] * 8
            for cc in range(8):
                ptbuf[b, cc, pl.ds(j * 16, 16)] = plsc.load_gather(
                    coords_v, [idxv + cc])

    def fire_wb(c, b):
        pltpu.async_copy(ptbuf.at[b], pt_out.at[pl.ds((row0 + c) * 8, 8)], wp_sem.at[b])

    def wait_wb(c, b):
        pltpu.make_async_copy(ptbuf.at[b], pt_out.at[pl.ds((row0 + c) * 8, 8)], wp_sem.at[b]).wait()

    fire_idx(0, 0)
    fire_idx(1, 1)
    pltpu.make_async_copy(coords_hbm, coords_v, tab_sem).wait()

    @pl.loop(0, (nch + 1) // 2)
    def _outer(g):
        for b in range(2):
            c = g * 2 + b

            @pl.when(c < nch)
            def _chunk():
                pltpu.make_async_copy(idx_hbm.at[row0 + c], idxbuf.at[b], idx_sem.at[b]).wait()

                @pl.when(c >= 2)
                def _slot_free():
                    wait_wb(c, b)

                pack(b)

                @pl.when(c + 2 < nch)
                def _prefetch():
                    fire_idx(c + 2, b)

                fire_wb(c, b)

    wait_wb(0, 0)
    wait_wb(0, 1)


def _sc_feat_body(r_slice, feats_hbm, idx_hbm, g_out,
                  idxbuf, gbuf, idx_sem, gg_sem, wg_sem):
    r_base, r_extra = r_slice // NW, r_slice % NW
    wid = lax.axis_index("s") * NC + lax.axis_index("c")
    nch = jnp.where(wid < r_extra, r_base + 1, r_base)
    row0 = r_base * wid + jnp.minimum(wid, r_extra)

    def fire_idx(c, bi):
        pltpu.async_copy(idx_hbm.at[row0 + c], idxbuf.at[bi], idx_sem.at[bi])

    def fire_gather(bi, bg):
        pltpu.async_copy(feats_hbm.at[idxbuf.at[bi]], gbuf.at[bg], gg_sem.at[bg])

    def wait_gather(bi, bg):
        pltpu.make_async_copy(feats_hbm.at[idxbuf.at[bi]], gbuf.at[bg], gg_sem.at[bg]).wait()

    def fire_wb(c, bg):
        pltpu.async_copy(gbuf.at[bg], g_out.at[pl.ds((row0 + c) * CH, CH)], wg_sem.at[bg])

    def wait_wb(c, bg):
        pltpu.make_async_copy(gbuf.at[bg], g_out.at[pl.ds((row0 + c) * CH, CH)], wg_sem.at[bg]).wait()

    fire_idx(0, 0)
    fire_idx(1, 1)
    fire_idx(2, 2)

    @pl.loop(0, (nch + 4) // 4)
    def _outer(g):
        for b in range(4):
            c = g * 4 + b

            @pl.when(c < nch)
            def _start():
                pltpu.make_async_copy(idx_hbm.at[row0 + c], idxbuf.at[b],
                                      idx_sem.at[b]).wait()

                @pl.when(c >= 2)
                def _slot_free():
                    wait_wb(c, b % 2)

                fire_gather(b, b % 2)

            @pl.when((c >= 1) & (c - 1 < nch))
            def _finish_prev():
                wait_gather((b + 3) % 4, (b + 1) % 2)

                @pl.when(c + 2 < nch)
                def _prefetch():
                    fire_idx(c + 2, (b + 2) % 4)

                fire_wb(c - 1, (b + 1) % 2)

    wait_wb(0, 0)
    wait_wb(0, 1)


def _sc_mesh():
    return plsc.VectorSubcoreMesh(core_axis_name="c", subcore_axis_name="s",
                                  num_cores=NC, num_subcores=NS)


def _make_sc_pack_coords():
    return pl.kernel(
        _sc_pack_coords_body,
        out_type=jax.ShapeDtypeStruct((R_TOT * 8, 128), jnp.float32),
        mesh=_sc_mesh(),
        compiler_params=pltpu.CompilerParams(needs_layout_passes=False),
        scratch_types=[
            pltpu.VMEM(((N + 1) * 8,), jnp.float32),
            pltpu.VMEM((2, CH), jnp.int32),
            pltpu.VMEM((2, 8, 128), jnp.float32),
            pltpu.SemaphoreType.DMA,
            pltpu.SemaphoreType.DMA((2,)),
            pltpu.SemaphoreType.DMA((2,)),
        ],
    )


def _make_sc_feat(r_slice):
    return pl.kernel(
        functools.partial(_sc_feat_body, r_slice),
        out_type=jax.ShapeDtypeStruct((r_slice * CH, IN_F), jnp.float32),
        mesh=_sc_mesh(),
        compiler_params=pltpu.CompilerParams(needs_layout_passes=False),
        scratch_types=[
            pltpu.VMEM((4, CH), jnp.int32),
            pltpu.VMEM((2, CH, IN_F), jnp.float32),
            pltpu.SemaphoreType.DMA((4,)),
            pltpu.SemaphoreType.DMA((2,)),
            pltpu.SemaphoreType.DMA((2,)),
        ],
    )


B = 400
EB = B * H


def _tc_body(g_ref, pt_ref, q_ref, c_ref, v_ref, o_ref):
    NCK = EB // CH
    QPC = CH // H
    S3 = q_ref[...][:, None, :] + c_ref[...][None, :, :]
    SF = S3.reshape(B * 16, 8)
    S2 = jnp.sum(SF * SF, axis=1, keepdims=True)
    row_q = lax.broadcasted_iota(jnp.int32, (QPC * 16, CH), 0) // 16
    col_q = lax.broadcasted_iota(jnp.int32, (QPC * 16, CH), 1) // H
    bigmask = jnp.where(row_q == col_q, 0.0, 1e9).astype(jnp.float32)

    Pt3 = pt_ref[...].reshape(NCK, 8, CH)
    wf = []
    for m in range(NCK):
        Ptm = Pt3[m]
        Sm = SF[m * QPC * 16:(m + 1) * QPC * 16, :]
        d2 = S2[m * QPC * 16:(m + 1) * QPC * 16, :] + bigmask
        for c in range(3):
            prow = Ptm[c:c + 1, :]
            d2 = d2 + prow * (prow - 2.0 * Sm[:, c:c + 1])
        Wm = jnp.maximum(1.0 - jnp.sqrt(jnp.maximum(d2, 0.0)) * (1.0 / KP_EXTENT),
                         0.0)
        Gm = g_ref[m * CH:(m + 1) * CH, :]
        wf.append(jnp.dot(Wm, Gm, preferred_element_type=jnp.float32))
    WF = jnp.stack(wf).reshape(NCK, QPC, 16, OUT_F)
    acc = jnp.zeros((B, OUT_F), jnp.float32)
    for k in range(N_KP):
        Fk = WF[:, :, k, :].reshape(B, IN_F)
        acc = acc + jnp.dot(Fk, v_ref[k], preferred_element_type=jnp.float32)
    o_ref[...] = acc


def _make_tc_compute(nq, sl):
    nblk = nq // B
    off = sl * nblk
    return pl.pallas_call(
        _tc_body,
        grid=(nblk,),
        in_specs=[
            pl.BlockSpec((EB, IN_F), lambda i: (i, 0)),
            pl.BlockSpec((EB // CH * 8, 128), lambda i: (i + off, 0)),
            pl.BlockSpec((B, 8), lambda i: (i + off, 0)),
            pl.BlockSpec((16, 8), lambda i: (0, 0)),
            pl.BlockSpec((N_KP, IN_F, OUT_F), lambda i: (0, 0, 0)),
        ],
        out_specs=pl.BlockSpec((B, OUT_F), lambda i: (i, 0)),
        out_shape=jax.ShapeDtypeStruct((nq, OUT_F), jnp.float32),
    )


def kernel(query_points, support_points, neighbors, x, K_points, K_values):
    idx = jnp.where(neighbors < 0, N, neighbors).astype(jnp.int32).reshape(R_TOT, CH)
    feats_tab = jnp.concatenate([x, jnp.zeros((1, IN_F), x.dtype)], axis=0)
    coords_tab = jnp.concatenate(
        [support_points, jnp.full((1, 3), 1e6, support_points.dtype)], axis=0)
    coords_flat = jnp.pad(coords_tab, ((0, 0), (0, 5))).reshape(-1)
    q_pad = jnp.pad(query_points, ((0, 0), (0, 5)))
    c_pad = jnp.pad(K_points, ((0, 1), (0, 5)),
                    constant_values=0.0).at[N_KP, :3].set(1e6)

    NSLC = 5
    r_slice = R_TOT // NSLC
    nq = N // NSLC
    pt = _make_sc_pack_coords()(coords_flat, idx)
    sc_call = _make_sc_feat(r_slice)
    outs = []
    for sl in range(NSLC):
        g = sc_call(feats_tab,
                    lax.slice_in_dim(idx, sl * r_slice, (sl + 1) * r_slice))
        outs.append(_make_tc_compute(nq, sl)(g, pt, q_pad, c_pad, K_values))
    return jnp.concatenate(outs, axis=0)
